# baseline (speedup 1.0000x reference)
import functools

import jax
import jax.numpy as jnp
import numpy as np
from jax import lax
from jax.experimental import pallas as pl
from jax.experimental.pallas import tpu as pltpu
from jax.experimental.pallas import tpu_sc as plsc

FM = 64
OUT_BITS = 16
PASS_STEPS = 3
VAR = 50000
CONST = 50000
OBJ = 8
E = 800000

BLK = 2000
NBLK = VAR // BLK

_NC = 2
_NS = 16
_LANES = 16
_FH = FM // _NC
_NPAD = 50048
_RPS = _NPAD // _NS
_EPAD = 819200
_EPS = _EPAD // _NS
_BC = 2560
_C = 128
_NCH = _BC // _C
_NBC = _EPS // _BC

_GDN = lax.GatherDimensionNumbers(offset_dims=(), collapsed_slice_dims=(0,),
                                  start_index_map=(0,))


def _lane_bcast(v, i):
    idx = lax.broadcast_in_dim(jnp.int32(i), (_LANES, 1), ())
    return lax.gather(v, idx, _GDN, (1,),
                      mode=lax.GatherScatterMode.PROMISE_IN_BOUNDS)


def _sc_spmm_body(tabl_hbm, tabr_hbm, src_hbm, dst_hbm, val_hbm, zeros_hbm,
                  out_hbm, shared, srcb, dstb, valb, rows0, rows1,
                  scat0, scat1, gsem0, gsem1, ssem0, ssem1):
    c = lax.axis_index("c")
    s = lax.axis_index("s")
    fcol = pl.multiple_of(c * _FH, _FH)
    my_rows = s * _RPS

    pltpu.sync_copy(zeros_hbm, shared.at[pl.ds(my_rows, _RPS)])
    plsc.subcore_barrier()

    ebase = s * _EPS

    def phase2(table_hbm):
        def gather_start(ck, rows, gsem):
            pos = pl.multiple_of(ck * _C, _C)
            return pltpu.async_copy(table_hbm.at[srcb.at[pl.ds(pos, _C)]],
                                    rows, gsem)

        def gather_wait(ck, rows, gsem):
            pos = pl.multiple_of(ck * _C, _C)
            pltpu.make_async_copy(table_hbm.at[srcb.at[pl.ds(pos, _C)]],
                                  rows, gsem).wait()

        def process(ck, rows, scat):
            pos = pl.multiple_of(ck * _C, _C)
            for j in range(_C // _LANES):
                sl = pl.ds(pos + j * _LANES, _LANES)
                scat[pl.ds(j * _LANES, _LANES)] = dstb[sl]
                vvm = valb[sl]
                for i in range(_LANES):
                    b16 = _lane_bcast(vvm, i)
                    r = j * _LANES + i
                    for q in range(_FH // _LANES):
                        fs = pl.ds(q * _LANES, _LANES)
                        rows[r, fs] = rows[r, fs] * b16

        def scatter_start(rows, scat, ssem):
            return pltpu.async_copy(rows, shared.at[scat], ssem, add=True)

        def scatter_wait(rows, scat, ssem):
            pltpu.make_async_copy(rows, shared.at[scat], ssem).wait()

        def pair_body(m, carry):
            c0, c1, c2, c3 = 2 * m, 2 * m + 1, 2 * m + 2, 2 * m + 3
            gather_wait(c0, rows0, gsem0)
            process(c0, rows0, scat0)
            scatter_start(rows0, scat0, ssem0)
            gather_wait(c1, rows1, gsem1)
            process(c1, rows1, scat1)
            scatter_start(rows1, scat1, ssem1)
            scatter_wait(rows0, scat0, ssem0)
            gather_start(c2, rows0, gsem0)
            scatter_wait(rows1, scat1, ssem1)
            gather_start(c3, rows1, gsem1)
            return carry

        def big_body(kb, carry):
            off = pl.multiple_of(ebase + kb * _BC, 8)
            pltpu.sync_copy(src_hbm.at[pl.ds(off, _BC)], srcb)
            pltpu.sync_copy(dst_hbm.at[pl.ds(off, _BC)], dstb)
            pltpu.sync_copy(val_hbm.at[pl.ds(off, _BC)], valb)
            gather_start(0, rows0, gsem0)
            gather_start(1, rows1, gsem1)
            carry = lax.fori_loop(0, _NCH // 2 - 1, pair_body, carry)
            gather_wait(_NCH - 2, rows0, gsem0)
            process(_NCH - 2, rows0, scat0)
            scatter_start(rows0, scat0, ssem0)
            gather_wait(_NCH - 1, rows1, gsem1)
            process(_NCH - 1, rows1, scat1)
            scatter_start(rows1, scat1, ssem1)
            scatter_wait(rows0, scat0, ssem0)
            scatter_wait(rows1, scat1, ssem1)
            return carry

        lax.fori_loop(0, _NBC, big_body, 0)

    @pl.when(c == 0)
    def _():
        phase2(tabl_hbm)

    @pl.when(c == 1)
    def _():
        phase2(tabr_hbm)

    plsc.subcore_barrier()
    pltpu.sync_copy(shared.at[pl.ds(my_rows, _RPS)],
                    out_hbm.at[pl.ds(my_rows, _RPS), pl.ds(fcol, _FH)])


@jax.jit
def _sc_spmm(tabl, tabr, src, dst, val, zeros_pad):
    mesh = plsc.VectorSubcoreMesh(core_axis_name="c", subcore_axis_name="s",
                                  num_cores=_NC, num_subcores=_NS)
    fn = pl.kernel(
        _sc_spmm_body,
        out_type=jax.ShapeDtypeStruct((_NPAD, FM), jnp.float32),
        mesh=mesh,
        scratch_types=[
            pltpu.VMEM_SHARED((_NPAD, _FH), jnp.float32),
            pltpu.VMEM((_BC,), jnp.int32),
            pltpu.VMEM((_BC,), jnp.int32),
            pltpu.VMEM((_BC,), jnp.float32),
            pltpu.VMEM((_C, _FH), jnp.float32),
            pltpu.VMEM((_C, _FH), jnp.float32),
            pltpu.VMEM((_C,), jnp.int32),
            pltpu.VMEM((_C,), jnp.int32),
            pltpu.SemaphoreType.DMA,
            pltpu.SemaphoreType.DMA,
            pltpu.SemaphoreType.DMA,
            pltpu.SemaphoreType.DMA,
        ],
        compiler_params=pltpu.CompilerParams(use_tc_tiling_on_sc=False),
        name="sc_spmm",
    )
    return fn(tabl, tabr, src, dst, val, zeros_pad)


def _stats_accum(i, h, sum_ref, ssq_ref):
    @pl.when(i == 0)
    def _():
        sum_ref[...] = jnp.zeros_like(sum_ref)
        ssq_ref[...] = jnp.zeros_like(ssq_ref)

    sum_ref[...] += jnp.sum(h, axis=0, keepdims=True)
    ssq_ref[...] += jnp.sum(h * h)[None, None]


def _dot(a, b):
    return jnp.dot(a, b, preferred_element_type=jnp.float32)


def _init_body(cond_ref, w1_ref, b1_ref, w2_ref, b2_ref,
               h_ref, sum_ref, ssq_ref):
    i = pl.program_id(0)
    x = cond_ref[...]
    h1 = jnp.maximum(x * w1_ref[...] + b1_ref[...], 0.0)
    h = _dot(h1, w2_ref[...]) + b2_ref[...]
    h_ref[...] = h
    _stats_accum(i, h, sum_ref, ssq_ref)


def _cons_obj_body(cons_ref, v2c_ref, var_ref, ocol_ref, oval_ref, obj_ref,
                   wa_ref, wb_ref, b1_ref, w2_ref, b2_ref,
                   owa_ref, owb_ref, ob1_ref, ow2_ref, ob2_ref,
                   h_ref, sum_ref, ssq_ref, obj_out_ref, acc_ref):
    i = pl.program_id(0)
    h1 = jnp.maximum(_dot(cons_ref[...], wa_ref[...])
                     + _dot(v2c_ref[...], wb_ref[...]) + b1_ref[...], 0.0)
    h = _dot(h1, w2_ref[...]) + b2_ref[...]
    h_ref[...] = h
    _stats_accum(i, h, sum_ref, ssq_ref)

    @pl.when(i == 0)
    def _():
        acc_ref[...] = jnp.zeros_like(acc_ref)

    sc = _scaled_onehot_t(ocol_ref, oval_ref)
    acc_ref[...] += _dot(sc, var_ref[...])

    @pl.when(i == pl.num_programs(0) - 1)
    def _():
        v2o = acc_ref[...]
        g1 = jnp.maximum(_dot(obj_ref[...], owa_ref[...])
                         + _dot(v2o, owb_ref[...]) + ob1_ref[...], 0.0)
        g = _dot(g1, ow2_ref[...]) + ob2_ref[...]
        gm = jnp.mean(g, axis=0, keepdims=True)
        ctr = g - gm
        scale = jnp.sqrt(jnp.mean(jnp.sum(ctr * ctr, axis=1)) + 1e-6)
        obj_out_ref[...] = ctr / scale


def _scaled_onehot_t(ocol_ref, oval_ref):
    ocol = ocol_ref[...].reshape(1, BLK)
    oval = oval_ref[...].reshape(1, BLK)
    rows = lax.broadcasted_iota(jnp.int32, (OBJ, BLK), 0)
    return jnp.where(rows == ocol, oval, 0.0)


def _var_body(var_ref, c2v_ref, ocol_ref, oval_ref, obj_ref,
              wa_ref, wb_ref, wc_ref, b1_ref, w2_ref, b2_ref,
              h_ref, sum_ref, ssq_ref):
    i = pl.program_id(0)
    sc = _scaled_onehot_t(ocol_ref, oval_ref)
    o2v = lax.dot_general(sc, obj_ref[...], (((0,), (0,)), ((), ())),
                          preferred_element_type=jnp.float32)
    h1 = jnp.maximum(_dot(var_ref[...], wa_ref[...])
                     + _dot(c2v_ref[...], wb_ref[...])
                     + _dot(o2v, wc_ref[...]) + b1_ref[...], 0.0)
    h = _dot(h1, w2_ref[...]) + b2_ref[...]
    h_ref[...] = h
    _stats_accum(i, h, sum_ref, ssq_ref)


def _pairnorm_consts(sum_ref, ssq_ref):
    n = float(VAR)
    m = sum_ref[...] / n
    ssq_centered = ssq_ref[0, 0] - n * jnp.sum(m * m)
    inv = lax.rsqrt(ssq_centered / n + 1e-6)
    return m, inv


def _norm_body(h_ref, sum_ref, ssq_ref, y_ref, yl_ref, yr_ref):
    m, inv = _pairnorm_consts(sum_ref, ssq_ref)
    y = (h_ref[...] - m) * inv
    y_ref[...] = y
    yl_ref[...] = y[:, :_FH]
    yr_ref[...] = y[:, _FH:]


def _head_body(h_ref, sum_ref, ssq_ref, w1_ref, b1_ref, w2_ref, b2_ref,
               noise_ref, pow_ref, v_ref, vl_ref, vr_ref, bits_ref, dec_ref):
    m, inv = _pairnorm_consts(sum_ref, ssq_ref)
    v = (h_ref[...] - m) * inv
    v_ref[...] = v
    vl_ref[...] = v[:, :_FH]
    vr_ref[...] = v[:, _FH:]
    u = jnp.maximum(_dot(v, w1_ref[...]) + b1_ref[...], 0.0)
    logits = _dot(u, w2_ref[...]) + b2_ref[...] + noise_ref[...]
    out = jax.nn.sigmoid(logits)
    bits_ref[...] = out
    dec_ref[...] = jnp.sum(out * pow_ref[...], axis=1, keepdims=True)


def _row_spec(width=FM):
    return pl.BlockSpec((BLK, width), lambda i: (i, 0))


def _const_spec(shape):
    return pl.BlockSpec(shape, lambda i: tuple(0 for _ in shape))


_IDX3_SPEC = pl.BlockSpec((1, 1, BLK), lambda i: (i, 0, 0))


def _stats_shapes():
    return [jax.ShapeDtypeStruct((VAR, FM), jnp.float32),
            jax.ShapeDtypeStruct((1, FM), jnp.float32),
            jax.ShapeDtypeStruct((1, 1), jnp.float32)]


def _stats_specs():
    return [_row_spec(), _const_spec((1, FM)), _const_spec((1, 1))]


@jax.jit
def _init_call(cond2, w1, b1, w2, b2):
    return pl.pallas_call(
        _init_body, grid=(NBLK,),
        in_specs=[_row_spec(1), _const_spec((1, FM)), _const_spec((1, FM)),
                  _const_spec((FM, FM)), _const_spec((1, FM))],
        out_shape=_stats_shapes(), out_specs=_stats_specs(),
    )(cond2, w1, b1, w2, b2)


@jax.jit
def _cons_obj_call(cons, v2c, variables, ocol3, oval3, objectives,
                   wa, wb, b1, w2, b2, owa, owb, ob1, ow2, ob2):
    return pl.pallas_call(
        _cons_obj_body, grid=(NBLK,),
        in_specs=[_row_spec(), _row_spec(), _row_spec(), _IDX3_SPEC,
                  _IDX3_SPEC, _const_spec((OBJ, FM)),
                  _const_spec((FM, FM)), _const_spec((FM, FM)),
                  _const_spec((1, FM)), _const_spec((FM, FM)),
                  _const_spec((1, FM)),
                  _const_spec((FM, FM)), _const_spec((FM, FM)),
                  _const_spec((1, FM)), _const_spec((FM, FM)),
                  _const_spec((1, FM))],
        out_shape=_stats_shapes() + [jax.ShapeDtypeStruct((OBJ, FM),
                                                          jnp.float32)],
        out_specs=_stats_specs() + [_const_spec((OBJ, FM))],
        scratch_shapes=[pltpu.VMEM((OBJ, FM), jnp.float32)],
    )(cons, v2c, variables, ocol3, oval3, objectives,
      wa, wb, b1, w2, b2, owa, owb, ob1, ow2, ob2)


@jax.jit
def _var_call(variables, c2v, ocol3, oval3, objectives,
              wa, wb, wc, b1, w2, b2):
    return pl.pallas_call(
        _var_body, grid=(NBLK,),
        in_specs=[_row_spec(), _row_spec(), _IDX3_SPEC, _IDX3_SPEC,
                  _const_spec((OBJ, FM)), _const_spec((FM, FM)),
                  _const_spec((FM, FM)), _const_spec((FM, FM)),
                  _const_spec((1, FM)), _const_spec((FM, FM)),
                  _const_spec((1, FM))],
        out_shape=_stats_shapes(), out_specs=_stats_specs(),
    )(variables, c2v, ocol3, oval3, objectives, wa, wb, wc, b1, w2, b2)


@jax.jit
def _norm_call(h, sm, sq):
    return pl.pallas_call(
        _norm_body, grid=(NBLK,),
        in_specs=[_row_spec(), _const_spec((1, FM)), _const_spec((1, 1))],
        out_shape=[jax.ShapeDtypeStruct((VAR, FM), jnp.float32),
                   jax.ShapeDtypeStruct((VAR, _FH), jnp.float32),
                   jax.ShapeDtypeStruct((VAR, _FH), jnp.float32)],
        out_specs=[_row_spec(), _row_spec(_FH), _row_spec(_FH)],
    )(h, sm, sq)


@jax.jit
def _head_call(h, sm, sq, w1, b1, w2, b2, noise, powers):
    return pl.pallas_call(
        _head_body, grid=(NBLK,),
        in_specs=[_row_spec(), _const_spec((1, FM)), _const_spec((1, 1)),
                  _const_spec((FM, FM)), _const_spec((1, FM)),
                  _const_spec((FM, OUT_BITS)), _const_spec((1, OUT_BITS)),
                  _row_spec(OUT_BITS), _const_spec((1, OUT_BITS))],
        out_shape=[jax.ShapeDtypeStruct((VAR, FM), jnp.float32),
                   jax.ShapeDtypeStruct((VAR, _FH), jnp.float32),
                   jax.ShapeDtypeStruct((VAR, _FH), jnp.float32),
                   jax.ShapeDtypeStruct((VAR, OUT_BITS), jnp.float32),
                   jax.ShapeDtypeStruct((VAR, 1), jnp.float32)],
        out_specs=[_row_spec(), _row_spec(_FH), _row_spec(_FH),
                   _row_spec(OUT_BITS), _row_spec(1)],
    )(h, sm, sq, w1, b1, w2, b2, noise, powers)


def kernel(adj_rows, adj_cols, adj_values, conditions_values, obj_rows,
           obj_cols, obj_values, params):
    f32 = jnp.float32
    spread = (jnp.arange(_EPAD - E, dtype=jnp.int32) * 8) % jnp.int32(VAR)
    adj_rows = jnp.concatenate([adj_rows.astype(jnp.int32), spread])
    adj_cols = jnp.concatenate([adj_cols.astype(jnp.int32), spread])
    adj_values = jnp.concatenate(
        [adj_values.astype(f32), jnp.zeros((_EPAD - E,), f32)])
    ocol3 = obj_cols.astype(jnp.int32).reshape(NBLK, 1, BLK)
    oval3 = obj_values.astype(f32).reshape(NBLK, 1, BLK)
    cond2 = conditions_values.astype(f32).reshape(CONST, 1)
    zeros_pad = jnp.zeros((_RPS, _FH), f32)
    powers = (2.0 ** jnp.arange(OUT_BITS, dtype=f32)).reshape(1, OUT_BITS)
    nkey = jax.random.key(42)
    nkeys = jnp.stack([jax.random.fold_in(nkey, i) for i in range(PASS_STEPS)])
    noises = jax.vmap(
        lambda k: jax.random.normal(k, (VAR, OUT_BITS), dtype=f32))(nkeys)

    def w(name):
        ww, bb = params[name]
        return ww.astype(f32), bb.astype(f32).reshape(1, -1)

    w_pc1, b_pc1 = w('pc1')
    w_pc2, b_pc2 = w('pc2')
    w_cu1, b_cu1 = w('cu1')
    w_cu2, b_cu2 = w('cu2')
    w_vu1, b_vu1 = w('vu1')
    w_vu2, b_vu2 = w('vu2')
    w_ou1, b_ou1 = w('ou1')
    w_ou2, b_ou2 = w('ou2')
    w_o1, b_o1 = w('o1')
    w_o2, b_o2 = w('o2')
    ca, cb = w_cu1[:FM], w_cu1[FM:]
    va, vb, vc = w_vu1[:FM], w_vu1[FM:2 * FM], w_vu1[2 * FM:]
    oa, ob = w_ou1[:FM], w_ou1[FM:]
    w_pc1 = w_pc1.reshape(1, FM)

    variables = jnp.ones((VAR, FM), f32)
    var_l = jnp.ones((VAR, _FH), f32)
    var_r = jnp.ones((VAR, _FH), f32)
    objectives = jnp.ones((OBJ, FM), f32)

    h0, sm0, sq0 = _init_call(cond2, w_pc1, b_pc1, w_pc2, b_pc2)
    constraints, cons_l, cons_r = _norm_call(h0, sm0, sq0)

    binary_outputs = []
    decimal_outputs = []
    for i in range(PASS_STEPS):
        noise = noises[i]
        v2c = _sc_spmm(var_l, var_r, adj_rows, adj_cols, adj_values,
                       zeros_pad)
        hc, smc, sqc, objectives = _cons_obj_call(
            constraints, v2c, variables, ocol3, oval3, objectives,
            ca, cb, b_cu1, w_cu2, b_cu2, oa, ob, b_ou1, w_ou2, b_ou2)
        constraints, cons_l, cons_r = _norm_call(hc, smc, sqc)
        c2v = _sc_spmm(cons_l, cons_r, adj_cols, adj_rows, adj_values,
                       zeros_pad)
        hv, smv, sqv = _var_call(variables, c2v, ocol3, oval3,
                                 objectives, va, vb, vc, b_vu1, w_vu2, b_vu2)
        variables, var_l, var_r, bits, dec = _head_call(
            hv, smv, sqv, w_o1, b_o1, w_o2, b_o2, noise, powers)
        binary_outputs.append(bits)
        decimal_outputs.append(dec)

    return (tuple(binary_outputs), tuple(decimal_outputs))


# speedup vs baseline: 1.0342x; 1.0342x over previous
import functools

import jax
import jax.numpy as jnp
import numpy as np
from jax import lax
from jax.experimental import pallas as pl
from jax.experimental.pallas import tpu as pltpu
from jax.experimental.pallas import tpu_sc as plsc

FM = 64
OUT_BITS = 16
PASS_STEPS = 3
VAR = 50000
CONST = 50000
OBJ = 8
E = 800000

BLK = 2000
NBLK = VAR // BLK

_NC = 2
_NS = 16
_LANES = 16
_FH = FM // _NC
_NPAD = 50048
_RPS = _NPAD // _NS
_EPAD = 819200
_EPS = _EPAD // _NS
_BC = 6400
_C = 128
_NCH = _BC // _C
_NBC = _EPS // _BC

_GDN = lax.GatherDimensionNumbers(offset_dims=(), collapsed_slice_dims=(0,),
                                  start_index_map=(0,))


def _lane_bcast(v, i):
    idx = lax.broadcast_in_dim(jnp.int32(i), (_LANES, 1), ())
    return lax.gather(v, idx, _GDN, (1,),
                      mode=lax.GatherScatterMode.PROMISE_IN_BOUNDS)


def _sc_spmm_body(tabl_hbm, tabr_hbm, src_hbm, dst_hbm, val_hbm, zeros_hbm,
                  out_hbm, shared, srcb, dstb, valb, rows0, rows1,
                  scat0, scat1, gsem0, gsem1, ssem0, ssem1):
    c = lax.axis_index("c")
    s = lax.axis_index("s")
    fcol = pl.multiple_of(c * _FH, _FH)
    my_rows = s * _RPS

    pltpu.sync_copy(zeros_hbm, shared.at[pl.ds(my_rows, _RPS)])
    plsc.subcore_barrier()

    ebase = s * _EPS

    def phase2(table_hbm):
        def gather_start(ck, rows, gsem):
            pos = pl.multiple_of(ck * _C, _C)
            return pltpu.async_copy(table_hbm.at[srcb.at[pl.ds(pos, _C)]],
                                    rows, gsem)

        def gather_wait(ck, rows, gsem):
            pos = pl.multiple_of(ck * _C, _C)
            pltpu.make_async_copy(table_hbm.at[srcb.at[pl.ds(pos, _C)]],
                                  rows, gsem).wait()

        def process(ck, rows, scat):
            pos = pl.multiple_of(ck * _C, _C)
            for j in range(_C // _LANES):
                sl = pl.ds(pos + j * _LANES, _LANES)
                scat[pl.ds(j * _LANES, _LANES)] = dstb[sl]
                vvm = valb[sl]
                for i in range(_LANES):
                    b16 = _lane_bcast(vvm, i)
                    r = j * _LANES + i
                    for q in range(_FH // _LANES):
                        fs = pl.ds(q * _LANES, _LANES)
                        rows[r, fs] = rows[r, fs] * b16

        def scatter_start(rows, scat, ssem):
            return pltpu.async_copy(rows, shared.at[scat], ssem, add=True)

        def scatter_wait(rows, scat, ssem):
            pltpu.make_async_copy(rows, shared.at[scat], ssem).wait()

        def pair_body(m, carry):
            c0, c1, c2, c3 = 2 * m, 2 * m + 1, 2 * m + 2, 2 * m + 3
            gather_wait(c0, rows0, gsem0)
            process(c0, rows0, scat0)
            scatter_start(rows0, scat0, ssem0)
            gather_wait(c1, rows1, gsem1)
            process(c1, rows1, scat1)
            scatter_start(rows1, scat1, ssem1)
            scatter_wait(rows0, scat0, ssem0)
            gather_start(c2, rows0, gsem0)
            scatter_wait(rows1, scat1, ssem1)
            gather_start(c3, rows1, gsem1)
            return carry

        def big_body(kb, carry):
            off = pl.multiple_of(ebase + kb * _BC, 8)
            pltpu.sync_copy(src_hbm.at[pl.ds(off, _BC)], srcb)
            pltpu.sync_copy(dst_hbm.at[pl.ds(off, _BC)], dstb)
            pltpu.sync_copy(val_hbm.at[pl.ds(off, _BC)], valb)
            gather_start(0, rows0, gsem0)
            gather_start(1, rows1, gsem1)
            carry = lax.fori_loop(0, _NCH // 2 - 1, pair_body, carry)
            gather_wait(_NCH - 2, rows0, gsem0)
            process(_NCH - 2, rows0, scat0)
            scatter_start(rows0, scat0, ssem0)
            gather_wait(_NCH - 1, rows1, gsem1)
            process(_NCH - 1, rows1, scat1)
            scatter_start(rows1, scat1, ssem1)
            scatter_wait(rows0, scat0, ssem0)
            scatter_wait(rows1, scat1, ssem1)
            return carry

        lax.fori_loop(0, _NBC, big_body, 0)

    @pl.when(c == 0)
    def _():
        phase2(tabl_hbm)

    @pl.when(c == 1)
    def _():
        phase2(tabr_hbm)

    plsc.subcore_barrier()
    pltpu.sync_copy(shared.at[pl.ds(my_rows, _RPS)],
                    out_hbm.at[pl.ds(my_rows, _RPS), pl.ds(fcol, _FH)])


@jax.jit
def _sc_spmm(tabl, tabr, src, dst, val, zeros_pad):
    mesh = plsc.VectorSubcoreMesh(core_axis_name="c", subcore_axis_name="s",
                                  num_cores=_NC, num_subcores=_NS)
    fn = pl.kernel(
        _sc_spmm_body,
        out_type=jax.ShapeDtypeStruct((_NPAD, FM), jnp.float32),
        mesh=mesh,
        scratch_types=[
            pltpu.VMEM_SHARED((_NPAD, _FH), jnp.float32),
            pltpu.VMEM((_BC,), jnp.int32),
            pltpu.VMEM((_BC,), jnp.int32),
            pltpu.VMEM((_BC,), jnp.float32),
            pltpu.VMEM((_C, _FH), jnp.float32),
            pltpu.VMEM((_C, _FH), jnp.float32),
            pltpu.VMEM((_C,), jnp.int32),
            pltpu.VMEM((_C,), jnp.int32),
            pltpu.SemaphoreType.DMA,
            pltpu.SemaphoreType.DMA,
            pltpu.SemaphoreType.DMA,
            pltpu.SemaphoreType.DMA,
        ],
        compiler_params=pltpu.CompilerParams(use_tc_tiling_on_sc=False),
        name="sc_spmm",
    )
    return fn(tabl, tabr, src, dst, val, zeros_pad)


def _stats_accum(i, h, sum_ref, ssq_ref):
    @pl.when(i == 0)
    def _():
        sum_ref[...] = jnp.zeros_like(sum_ref)
        ssq_ref[...] = jnp.zeros_like(ssq_ref)

    sum_ref[...] += jnp.sum(h, axis=0, keepdims=True)
    ssq_ref[...] += jnp.sum(h * h)[None, None]


def _dot(a, b):
    return jnp.dot(a, b, preferred_element_type=jnp.float32)


def _init_body(cond_ref, w1_ref, b1_ref, w2_ref, b2_ref,
               h_ref, sum_ref, ssq_ref):
    i = pl.program_id(0)
    x = cond_ref[...]
    h1 = jnp.maximum(x * w1_ref[...] + b1_ref[...], 0.0)
    h = _dot(h1, w2_ref[...]) + b2_ref[...]
    h_ref[...] = h
    _stats_accum(i, h, sum_ref, ssq_ref)


def _cons_obj_body(cons_ref, v2c_ref, var_ref, ocol_ref, oval_ref, obj_ref,
                   wa_ref, wb_ref, b1_ref, w2_ref, b2_ref,
                   owa_ref, owb_ref, ob1_ref, ow2_ref, ob2_ref,
                   h_ref, sum_ref, ssq_ref, obj_out_ref, acc_ref):
    i = pl.program_id(0)
    h1 = jnp.maximum(_dot(cons_ref[...], wa_ref[...])
                     + _dot(v2c_ref[...], wb_ref[...]) + b1_ref[...], 0.0)
    h = _dot(h1, w2_ref[...]) + b2_ref[...]
    h_ref[...] = h
    _stats_accum(i, h, sum_ref, ssq_ref)

    @pl.when(i == 0)
    def _():
        acc_ref[...] = jnp.zeros_like(acc_ref)

    sc = _scaled_onehot_t(ocol_ref, oval_ref)
    acc_ref[...] += _dot(sc, var_ref[...])

    @pl.when(i == pl.num_programs(0) - 1)
    def _():
        v2o = acc_ref[...]
        g1 = jnp.maximum(_dot(obj_ref[...], owa_ref[...])
                         + _dot(v2o, owb_ref[...]) + ob1_ref[...], 0.0)
        g = _dot(g1, ow2_ref[...]) + ob2_ref[...]
        gm = jnp.mean(g, axis=0, keepdims=True)
        ctr = g - gm
        scale = jnp.sqrt(jnp.mean(jnp.sum(ctr * ctr, axis=1)) + 1e-6)
        obj_out_ref[...] = ctr / scale


def _scaled_onehot_t(ocol_ref, oval_ref):
    ocol = ocol_ref[...].reshape(1, BLK)
    oval = oval_ref[...].reshape(1, BLK)
    rows = lax.broadcasted_iota(jnp.int32, (OBJ, BLK), 0)
    return jnp.where(rows == ocol, oval, 0.0)


def _var_body(var_ref, c2v_ref, ocol_ref, oval_ref, obj_ref,
              wa_ref, wb_ref, wc_ref, b1_ref, w2_ref, b2_ref,
              h_ref, sum_ref, ssq_ref):
    i = pl.program_id(0)
    sc = _scaled_onehot_t(ocol_ref, oval_ref)
    o2v = lax.dot_general(sc, obj_ref[...], (((0,), (0,)), ((), ())),
                          preferred_element_type=jnp.float32)
    h1 = jnp.maximum(_dot(var_ref[...], wa_ref[...])
                     + _dot(c2v_ref[...], wb_ref[...])
                     + _dot(o2v, wc_ref[...]) + b1_ref[...], 0.0)
    h = _dot(h1, w2_ref[...]) + b2_ref[...]
    h_ref[...] = h
    _stats_accum(i, h, sum_ref, ssq_ref)


def _pairnorm_consts(sum_ref, ssq_ref):
    n = float(VAR)
    m = sum_ref[...] / n
    ssq_centered = ssq_ref[0, 0] - n * jnp.sum(m * m)
    inv = lax.rsqrt(ssq_centered / n + 1e-6)
    return m, inv


def _norm_body(h_ref, sum_ref, ssq_ref, y_ref, yl_ref, yr_ref):
    m, inv = _pairnorm_consts(sum_ref, ssq_ref)
    y = (h_ref[...] - m) * inv
    y_ref[...] = y
    yl_ref[...] = y[:, :_FH]
    yr_ref[...] = y[:, _FH:]


def _head_body(h_ref, sum_ref, ssq_ref, w1_ref, b1_ref, w2_ref, b2_ref,
               noise_ref, pow_ref, v_ref, vl_ref, vr_ref, bits_ref, dec_ref):
    m, inv = _pairnorm_consts(sum_ref, ssq_ref)
    v = (h_ref[...] - m) * inv
    v_ref[...] = v
    vl_ref[...] = v[:, :_FH]
    vr_ref[...] = v[:, _FH:]
    u = jnp.maximum(_dot(v, w1_ref[...]) + b1_ref[...], 0.0)
    logits = _dot(u, w2_ref[...]) + b2_ref[...] + noise_ref[...]
    out = jax.nn.sigmoid(logits)
    bits_ref[...] = out
    dec_ref[...] = jnp.sum(out * pow_ref[...], axis=1, keepdims=True)


def _row_spec(width=FM):
    return pl.BlockSpec((BLK, width), lambda i: (i, 0))


def _const_spec(shape):
    return pl.BlockSpec(shape, lambda i: tuple(0 for _ in shape))


_IDX3_SPEC = pl.BlockSpec((1, 1, BLK), lambda i: (i, 0, 0))


def _stats_shapes():
    return [jax.ShapeDtypeStruct((VAR, FM), jnp.float32),
            jax.ShapeDtypeStruct((1, FM), jnp.float32),
            jax.ShapeDtypeStruct((1, 1), jnp.float32)]


def _stats_specs():
    return [_row_spec(), _const_spec((1, FM)), _const_spec((1, 1))]


@jax.jit
def _init_call(cond2, w1, b1, w2, b2):
    return pl.pallas_call(
        _init_body, grid=(NBLK,),
        in_specs=[_row_spec(1), _const_spec((1, FM)), _const_spec((1, FM)),
                  _const_spec((FM, FM)), _const_spec((1, FM))],
        out_shape=_stats_shapes(), out_specs=_stats_specs(),
    )(cond2, w1, b1, w2, b2)


@jax.jit
def _cons_obj_call(cons, v2c, variables, ocol3, oval3, objectives,
                   wa, wb, b1, w2, b2, owa, owb, ob1, ow2, ob2):
    return pl.pallas_call(
        _cons_obj_body, grid=(NBLK,),
        in_specs=[_row_spec(), _row_spec(), _row_spec(), _IDX3_SPEC,
                  _IDX3_SPEC, _const_spec((OBJ, FM)),
                  _const_spec((FM, FM)), _const_spec((FM, FM)),
                  _const_spec((1, FM)), _const_spec((FM, FM)),
                  _const_spec((1, FM)),
                  _const_spec((FM, FM)), _const_spec((FM, FM)),
                  _const_spec((1, FM)), _const_spec((FM, FM)),
                  _const_spec((1, FM))],
        out_shape=_stats_shapes() + [jax.ShapeDtypeStruct((OBJ, FM),
                                                          jnp.float32)],
        out_specs=_stats_specs() + [_const_spec((OBJ, FM))],
        scratch_shapes=[pltpu.VMEM((OBJ, FM), jnp.float32)],
    )(cons, v2c, variables, ocol3, oval3, objectives,
      wa, wb, b1, w2, b2, owa, owb, ob1, ow2, ob2)


@jax.jit
def _var_call(variables, c2v, ocol3, oval3, objectives,
              wa, wb, wc, b1, w2, b2):
    return pl.pallas_call(
        _var_body, grid=(NBLK,),
        in_specs=[_row_spec(), _row_spec(), _IDX3_SPEC, _IDX3_SPEC,
                  _const_spec((OBJ, FM)), _const_spec((FM, FM)),
                  _const_spec((FM, FM)), _const_spec((FM, FM)),
                  _const_spec((1, FM)), _const_spec((FM, FM)),
                  _const_spec((1, FM))],
        out_shape=_stats_shapes(), out_specs=_stats_specs(),
    )(variables, c2v, ocol3, oval3, objectives, wa, wb, wc, b1, w2, b2)


@jax.jit
def _norm_call(h, sm, sq):
    return pl.pallas_call(
        _norm_body, grid=(NBLK,),
        in_specs=[_row_spec(), _const_spec((1, FM)), _const_spec((1, 1))],
        out_shape=[jax.ShapeDtypeStruct((VAR, FM), jnp.float32),
                   jax.ShapeDtypeStruct((VAR, _FH), jnp.float32),
                   jax.ShapeDtypeStruct((VAR, _FH), jnp.float32)],
        out_specs=[_row_spec(), _row_spec(_FH), _row_spec(_FH)],
    )(h, sm, sq)


@jax.jit
def _head_call(h, sm, sq, w1, b1, w2, b2, noise, powers):
    return pl.pallas_call(
        _head_body, grid=(NBLK,),
        in_specs=[_row_spec(), _const_spec((1, FM)), _const_spec((1, 1)),
                  _const_spec((FM, FM)), _const_spec((1, FM)),
                  _const_spec((FM, OUT_BITS)), _const_spec((1, OUT_BITS)),
                  _row_spec(OUT_BITS), _const_spec((1, OUT_BITS))],
        out_shape=[jax.ShapeDtypeStruct((VAR, FM), jnp.float32),
                   jax.ShapeDtypeStruct((VAR, _FH), jnp.float32),
                   jax.ShapeDtypeStruct((VAR, _FH), jnp.float32),
                   jax.ShapeDtypeStruct((VAR, OUT_BITS), jnp.float32),
                   jax.ShapeDtypeStruct((VAR, 1), jnp.float32)],
        out_specs=[_row_spec(), _row_spec(_FH), _row_spec(_FH),
                   _row_spec(OUT_BITS), _row_spec(1)],
    )(h, sm, sq, w1, b1, w2, b2, noise, powers)


def kernel(adj_rows, adj_cols, adj_values, conditions_values, obj_rows,
           obj_cols, obj_values, params):
    f32 = jnp.float32
    spread = (jnp.arange(_EPAD - E, dtype=jnp.int32) * 8) % jnp.int32(VAR)
    adj_rows = jnp.concatenate([adj_rows.astype(jnp.int32), spread])
    adj_cols = jnp.concatenate([adj_cols.astype(jnp.int32), spread])
    adj_values = jnp.concatenate(
        [adj_values.astype(f32), jnp.zeros((_EPAD - E,), f32)])
    ocol3 = obj_cols.astype(jnp.int32).reshape(NBLK, 1, BLK)
    oval3 = obj_values.astype(f32).reshape(NBLK, 1, BLK)
    cond2 = conditions_values.astype(f32).reshape(CONST, 1)
    zeros_pad = jnp.zeros((_RPS, _FH), f32)
    powers = (2.0 ** jnp.arange(OUT_BITS, dtype=f32)).reshape(1, OUT_BITS)
    nkey = jax.random.key(42)
    nkeys = jnp.stack([jax.random.fold_in(nkey, i) for i in range(PASS_STEPS)])
    noises = jax.vmap(
        lambda k: jax.random.normal(k, (VAR, OUT_BITS), dtype=f32))(nkeys)

    def w(name):
        ww, bb = params[name]
        return ww.astype(f32), bb.astype(f32).reshape(1, -1)

    w_pc1, b_pc1 = w('pc1')
    w_pc2, b_pc2 = w('pc2')
    w_cu1, b_cu1 = w('cu1')
    w_cu2, b_cu2 = w('cu2')
    w_vu1, b_vu1 = w('vu1')
    w_vu2, b_vu2 = w('vu2')
    w_ou1, b_ou1 = w('ou1')
    w_ou2, b_ou2 = w('ou2')
    w_o1, b_o1 = w('o1')
    w_o2, b_o2 = w('o2')
    ca, cb = w_cu1[:FM], w_cu1[FM:]
    va, vb, vc = w_vu1[:FM], w_vu1[FM:2 * FM], w_vu1[2 * FM:]
    oa, ob = w_ou1[:FM], w_ou1[FM:]
    w_pc1 = w_pc1.reshape(1, FM)

    variables = jnp.ones((VAR, FM), f32)
    var_l = jnp.ones((VAR, _FH), f32)
    var_r = jnp.ones((VAR, _FH), f32)
    objectives = jnp.ones((OBJ, FM), f32)

    h0, sm0, sq0 = _init_call(cond2, w_pc1, b_pc1, w_pc2, b_pc2)
    constraints, cons_l, cons_r = _norm_call(h0, sm0, sq0)

    binary_outputs = []
    decimal_outputs = []
    for i in range(PASS_STEPS):
        noise = noises[i]
        v2c = _sc_spmm(var_l, var_r, adj_rows, adj_cols, adj_values,
                       zeros_pad)
        hc, smc, sqc, objectives = _cons_obj_call(
            constraints, v2c, variables, ocol3, oval3, objectives,
            ca, cb, b_cu1, w_cu2, b_cu2, oa, ob, b_ou1, w_ou2, b_ou2)
        constraints, cons_l, cons_r = _norm_call(hc, smc, sqc)
        c2v = _sc_spmm(cons_l, cons_r, adj_cols, adj_rows, adj_values,
                       zeros_pad)
        hv, smv, sqv = _var_call(variables, c2v, ocol3, oval3,
                                 objectives, va, vb, vc, b_vu1, w_vu2, b_vu2)
        variables, var_l, var_r, bits, dec = _head_call(
            hv, smv, sqv, w_o1, b_o1, w_o2, b_o2, noise, powers)
        binary_outputs.append(bits)
        decimal_outputs.append(dec)

    return (tuple(binary_outputs), tuple(decimal_outputs))
